# Initial kernel scaffold; baseline (speedup 1.0000x reference)
import jax
import jax.numpy as jnp
from jax.experimental import pallas as pl


def kernel(x, layer_index, pe):
    raise NotImplementedError("write your pallas kernel here")


# speedup vs baseline: 2.7132x; 2.7132x over previous
import functools

import jax
import jax.numpy as jnp
from jax import lax
from jax.experimental import pallas as pl
from jax.experimental.pallas import tpu as pltpu
from jax.experimental.pallas import tpu_sc as plsc

D_MODEL = 128
LANES = 16
NUM_CORES = 2
NUM_SUBCORES = 16
NUM_WORKERS = NUM_CORES * NUM_SUBCORES
CHUNK = 128


@functools.partial(jax.jit, static_argnames=())
def _run(x, layer_index, pe2d):
    n = x.shape[0]
    rows_per_w = n // NUM_WORKERS
    chunks = rows_per_w // CHUNK
    mesh = plsc.VectorSubcoreMesh(core_axis_name="c", subcore_axis_name="s")

    @functools.partial(
        pl.kernel,
        mesh=mesh,
        out_type=jax.ShapeDtypeStruct((n, D_MODEL), jnp.float32),
        scratch_types=[
            pltpu.VMEM((CHUNK,), jnp.int32),
            pltpu.VMEM((CHUNK, D_MODEL), jnp.float32),
            pltpu.VMEM((CHUNK, D_MODEL), jnp.float32),
            pltpu.SemaphoreType.DMA,
        ],
    )
    def k(x_hbm, idx_hbm, pe_hbm, out_hbm, idx_v, x_v, pe_v, sem):
        wid = lax.axis_index("s") * NUM_CORES + lax.axis_index("c")

        def chunk_body(g, carry):
            base = wid * rows_per_w + g * CHUNK
            pltpu.sync_copy(idx_hbm.at[pl.ds(base, CHUNK)], idx_v)
            gather = pltpu.async_copy(pe_hbm.at[idx_v], pe_v, sem)
            pltpu.sync_copy(x_hbm.at[pl.ds(base, CHUNK)], x_v)
            gather.wait()

            def row_body(r, c2):
                def grp(j, c3):
                    sl = pl.ds(j * LANES, LANES)
                    x_v[r, sl] = x_v[r, sl] + pe_v[r, sl]
                    return c3

                return lax.fori_loop(0, D_MODEL // LANES, grp, c2)

            lax.fori_loop(0, CHUNK, row_body, carry)
            pltpu.sync_copy(x_v, out_hbm.at[pl.ds(base, CHUNK)])
            return carry

        lax.fori_loop(0, chunks, chunk_body, 0)

    return k(x, layer_index, pe2d)


def kernel(x, layer_index, pe):
    pe2d = pe.reshape(pe.shape[0], pe.shape[-1])
    return _run(x, layer_index, pe2d)


# speedup vs baseline: 2.9166x; 1.0750x over previous
import functools

import jax
import jax.numpy as jnp
from jax import lax
from jax.experimental import pallas as pl
from jax.experimental.pallas import tpu as pltpu
from jax.experimental.pallas import tpu_sc as plsc

D_MODEL = 128
LANES = 16
NUM_CORES = 2
NUM_SUBCORES = 16
NUM_WORKERS = NUM_CORES * NUM_SUBCORES
CHUNK = 128


@jax.jit
def _run(x, idx2d, pe2d):
    n = x.shape[0]
    rows_per_w = n // NUM_WORKERS
    chunks = rows_per_w // CHUNK
    mesh = plsc.VectorSubcoreMesh(core_axis_name="c", subcore_axis_name="s")

    @functools.partial(
        pl.kernel,
        mesh=mesh,
        out_type=jax.ShapeDtypeStruct((n, D_MODEL), jnp.float32),
        scratch_types=[
            pltpu.VMEM((chunks, CHUNK), jnp.int32),
            pltpu.VMEM((CHUNK, D_MODEL), jnp.float32),
            pltpu.VMEM((CHUNK, D_MODEL), jnp.float32),
            pltpu.VMEM((CHUNK, D_MODEL), jnp.float32),
            pltpu.VMEM((CHUNK, D_MODEL), jnp.float32),
            pltpu.VMEM((CHUNK, D_MODEL), jnp.float32),
            pltpu.VMEM((CHUNK, D_MODEL), jnp.float32),
            pltpu.SemaphoreType.DMA,
            pltpu.SemaphoreType.DMA,
            pltpu.SemaphoreType.DMA,
            pltpu.SemaphoreType.DMA,
            pltpu.SemaphoreType.DMA,
            pltpu.SemaphoreType.DMA,
        ],
    )
    def k(x_hbm, idx_hbm, pe_hbm, out_hbm,
          idxs, x0, x1, p0, p1, o0, o1, sx0, sx1, sp0, sp1, so0, so1):
        wid = lax.axis_index("s") * NUM_CORES + lax.axis_index("c")
        xb, pb, ob = (x0, x1), (p0, p1), (o0, o1)
        sx, sp, so = (sx0, sx1), (sp0, sp1), (so0, so1)

        pltpu.sync_copy(idx_hbm.at[pl.ds(wid * chunks, chunks)], idxs)

        def start_in(g, b):
            base = wid * rows_per_w + g * CHUNK
            pltpu.make_async_copy(
                x_hbm.at[pl.ds(base, CHUNK)], xb[b], sx[b]).start()
            pltpu.make_async_copy(
                pe_hbm.at[idxs.at[g]], pb[b], sp[b]).start()

        def wait_in(g, b):
            base = wid * rows_per_w + g * CHUNK
            pltpu.make_async_copy(
                x_hbm.at[pl.ds(base, CHUNK)], xb[b], sx[b]).wait()
            pltpu.make_async_copy(
                pe_hbm.at[idxs.at[g]], pb[b], sp[b]).wait()

        def start_out(g, b):
            base = wid * rows_per_w + g * CHUNK
            pltpu.make_async_copy(
                ob[b], out_hbm.at[pl.ds(base, CHUNK)], so[b]).start()

        def wait_out(g, b):
            base = wid * rows_per_w + g * CHUNK
            pltpu.make_async_copy(
                ob[b], out_hbm.at[pl.ds(base, CHUNK)], so[b]).wait()

        start_in(0, 0)
        start_in(1, 1)

        def step(g, b):
            wait_in(g, b)

            @pl.when(g >= 2)
            def _():
                wait_out(g - 2, b)

            def row_body(r, carry):
                for j in range(D_MODEL // LANES):
                    sl = pl.ds(j * LANES, LANES)
                    ob[b][r, sl] = xb[b][r, sl] + pb[b][r, sl]
                return carry

            lax.fori_loop(0, CHUNK, row_body, 0)
            start_out(g, b)

            @pl.when(g + 2 < chunks)
            def _():
                start_in(g + 2, b)

        def pair(h, carry):
            step(2 * h, 0)
            step(2 * h + 1, 1)
            return carry

        lax.fori_loop(0, chunks // 2, pair, 0)
        wait_out(chunks - 2, 0)
        wait_out(chunks - 1, 1)

    return k(x, idx2d, pe2d)


def kernel(x, layer_index, pe):
    pe2d = pe.reshape(pe.shape[0], pe.shape[-1])
    idx2d = layer_index.reshape(layer_index.shape[0] // CHUNK, CHUNK)
    return _run(x, idx2d, pe2d)


# speedup vs baseline: 8.0635x; 2.7647x over previous
import functools

import jax
import jax.numpy as jnp
from jax import lax
from jax.experimental import pallas as pl
from jax.experimental.pallas import tpu as pltpu
from jax.experimental.pallas import tpu_sc as plsc

D_MODEL = 128
LANES = 16
NUM_CORES = 2
NUM_SUBCORES = 16
NUM_WORKERS = NUM_CORES * NUM_SUBCORES
CHUNK = 128


@jax.jit
def _run(x, idx2d, pe2d):
    n = x.shape[0]
    rows_per_w = n // NUM_WORKERS
    chunks = rows_per_w // CHUNK
    mesh = plsc.VectorSubcoreMesh(core_axis_name="c", subcore_axis_name="s")

    @functools.partial(
        pl.kernel,
        mesh=mesh,
        out_type=jax.ShapeDtypeStruct((n, D_MODEL), jnp.float32),
        scratch_types=[
            pltpu.VMEM_SHARED((100, D_MODEL), jnp.float32),
            pltpu.VMEM((chunks, CHUNK), jnp.int32),
            pltpu.VMEM((CHUNK, D_MODEL), jnp.float32),
            pltpu.VMEM((CHUNK, D_MODEL), jnp.float32),
            pltpu.VMEM((CHUNK, D_MODEL), jnp.float32),
            pltpu.VMEM((CHUNK, D_MODEL), jnp.float32),
            pltpu.VMEM((CHUNK, D_MODEL), jnp.float32),
            pltpu.VMEM((CHUNK, D_MODEL), jnp.float32),
            pltpu.SemaphoreType.DMA,
            pltpu.SemaphoreType.DMA,
            pltpu.SemaphoreType.DMA,
            pltpu.SemaphoreType.DMA,
            pltpu.SemaphoreType.DMA,
            pltpu.SemaphoreType.DMA,
        ],
    )
    def k(x_hbm, idx_hbm, pe_hbm, out_hbm,
          pe_sh, idxs, x0, x1, p0, p1, o0, o1, sx0, sx1, sp0, sp1, so0, so1):
        wid = lax.axis_index("s") * NUM_CORES + lax.axis_index("c")
        xb, pb, ob = (x0, x1), (p0, p1), (o0, o1)
        sx, sp, so = (sx0, sx1), (sp0, sp1), (so0, so1)

        @pl.when(lax.axis_index("s") == 0)
        def _():
            pltpu.sync_copy(pe_hbm, pe_sh)

        pltpu.sync_copy(idx_hbm.at[pl.ds(wid * chunks, chunks)], idxs)
        plsc.subcore_barrier()

        def start_in(g, b):
            base = wid * rows_per_w + g * CHUNK
            pltpu.make_async_copy(
                x_hbm.at[pl.ds(base, CHUNK)], xb[b], sx[b]).start()
            pltpu.make_async_copy(
                pe_sh.at[idxs.at[g]], pb[b], sp[b]).start()

        def wait_in(g, b):
            base = wid * rows_per_w + g * CHUNK
            pltpu.make_async_copy(
                x_hbm.at[pl.ds(base, CHUNK)], xb[b], sx[b]).wait()
            pltpu.make_async_copy(
                pe_sh.at[idxs.at[g]], pb[b], sp[b]).wait()

        def start_out(g, b):
            base = wid * rows_per_w + g * CHUNK
            pltpu.make_async_copy(
                ob[b], out_hbm.at[pl.ds(base, CHUNK)], so[b]).start()

        def wait_out(g, b):
            base = wid * rows_per_w + g * CHUNK
            pltpu.make_async_copy(
                ob[b], out_hbm.at[pl.ds(base, CHUNK)], so[b]).wait()

        start_in(0, 0)
        start_in(1, 1)

        def step(g, b):
            wait_in(g, b)

            @pl.when(g >= 2)
            def _():
                wait_out(g - 2, b)

            def row_body(r, carry):
                for j in range(D_MODEL // LANES):
                    sl = pl.ds(j * LANES, LANES)
                    ob[b][r, sl] = xb[b][r, sl] + pb[b][r, sl]
                return carry

            lax.fori_loop(0, CHUNK, row_body, 0)
            start_out(g, b)

            @pl.when(g + 2 < chunks)
            def _():
                start_in(g + 2, b)

        def pair(h, carry):
            step(2 * h, 0)
            step(2 * h + 1, 1)
            return carry

        lax.fori_loop(0, chunks // 2, pair, 0)
        wait_out(chunks - 2, 0)
        wait_out(chunks - 1, 1)

    return k(x, idx2d, pe2d)


def kernel(x, layer_index, pe):
    pe2d = pe.reshape(pe.shape[0], pe.shape[-1])
    idx2d = layer_index.reshape(layer_index.shape[0] // CHUNK, CHUNK)
    return _run(x, idx2d, pe2d)


# speedup vs baseline: 8.7239x; 1.0819x over previous
import functools

import jax
import jax.numpy as jnp
from jax import lax
from jax.experimental import pallas as pl
from jax.experimental.pallas import tpu as pltpu
from jax.experimental.pallas import tpu_sc as plsc

D_MODEL = 128
NUM_CORES = 2
NUM_SUBCORES = 16
NUM_WORKERS = NUM_CORES * NUM_SUBCORES
CHUNK = 128
NBUF = 4


@jax.jit
def _run(x, idx2d, pe2d):
    n = x.shape[0]
    rows_per_w = n // NUM_WORKERS
    chunks = rows_per_w // CHUNK
    mesh = plsc.VectorSubcoreMesh(core_axis_name="c", subcore_axis_name="s")

    @functools.partial(
        pl.kernel,
        mesh=mesh,
        out_type=jax.ShapeDtypeStruct((n, D_MODEL), jnp.float32),
        scratch_types=[
            pltpu.VMEM_SHARED((100, D_MODEL), jnp.float32),
            pltpu.VMEM((chunks, CHUNK), jnp.int32),
        ]
        + [pltpu.VMEM((CHUNK, D_MODEL), jnp.float32) for _ in range(NBUF)]
        + [pltpu.SemaphoreType.DMA for _ in range(3 * NBUF)],
    )
    def k(x_hbm, idx_hbm, pe_hbm, out_hbm, pe_sh, idxs, *rest):
        xb = rest[:NBUF]
        sin = rest[NBUF:2 * NBUF]
        sadd = rest[2 * NBUF:3 * NBUF]
        sout = rest[3 * NBUF:4 * NBUF]
        wid = lax.axis_index("s") * NUM_CORES + lax.axis_index("c")

        @pl.when(lax.axis_index("s") == 0)
        def _():
            pltpu.sync_copy(pe_hbm, pe_sh)

        pltpu.sync_copy(idx_hbm.at[pl.ds(wid * chunks, chunks)], idxs)
        plsc.subcore_barrier()

        def in_copy(g, b):
            base = wid * rows_per_w + g * CHUNK
            return pltpu.make_async_copy(
                x_hbm.at[pl.ds(base, CHUNK)], xb[b], sin[b])

        def add_copy(g, b):
            return pltpu.make_async_copy(pe_sh.at[idxs.at[g]], xb[b], sadd[b])

        def out_copy(g, b):
            base = wid * rows_per_w + g * CHUNK
            return pltpu.make_async_copy(
                xb[b], out_hbm.at[pl.ds(base, CHUNK)], sout[b])

        in_copy(0, 0).start()
        in_copy(1, 1).start()

        def step(g, b):
            b2 = (b + 2) % NBUF

            @pl.when(g + 2 < chunks)
            def _():
                @pl.when(g >= 2)
                def _():
                    out_copy(g - 2, b2).wait()

                in_copy(g + 2, b2).start()

            in_copy(g, b).wait()
            add_copy(g, b).start(add=True)
            add_copy(g, b).wait()
            out_copy(g, b).start()

        def quad(h, carry):
            for u in range(NBUF):
                step(NBUF * h + u, u)
            return carry

        lax.fori_loop(0, chunks // NBUF, quad, 0)
        for u in range(NBUF):
            out_copy(chunks - NBUF + u, (chunks - NBUF + u) % NBUF).wait()

    return k(x, idx2d, pe2d)


def kernel(x, layer_index, pe):
    pe2d = pe.reshape(pe.shape[0], pe.shape[-1])
    idx2d = layer_index.reshape(layer_index.shape[0] // CHUNK, CHUNK)
    return _run(x, idx2d, pe2d)
